# Initial kernel scaffold; baseline (speedup 1.0000x reference)
import jax
import jax.numpy as jnp
from jax.experimental import pallas as pl


def kernel(inputs, embedding):
    raise NotImplementedError("write your pallas kernel here")


# speedup vs baseline: 6.1869x; 6.1869x over previous
import functools

import jax
import jax.numpy as jnp
from jax import lax
from jax.experimental import pallas as pl
from jax.experimental.pallas import tpu as pltpu
from jax.experimental.pallas import tpu_sc as plsc

N = 8192
K = 8192
D = 256
RB = 256
NBLK = N // RB

NC = 2
NS = 16
NW = NC * NS
BPW = N // NW


def _onehot_body(idx_ref, out_ref, cnt_ref):
    r = pl.program_id(0)
    idxb = idx_ref[...]
    iot = lax.broadcasted_iota(jnp.int32, (RB, K), 1)
    oh = jnp.where(iot == idxb, jnp.float32(1.0), jnp.float32(0.0))
    out_ref[...] = oh

    @pl.when(r == 0)
    def _():
        cnt_ref[...] = jnp.zeros((1, K), jnp.float32)

    cnt_ref[...] += jnp.sum(oh, axis=0, keepdims=True)


_onehot = pl.pallas_call(
    _onehot_body,
    grid=(NBLK,),
    in_specs=[pl.BlockSpec((RB, 1), lambda r: (r, 0))],
    out_specs=[
        pl.BlockSpec((RB, K), lambda r: (r, 0)),
        pl.BlockSpec((1, K), lambda r: (0, 0)),
    ],
    out_shape=[
        jax.ShapeDtypeStruct((N, K), jnp.float32),
        jax.ShapeDtypeStruct((1, K), jnp.float32),
    ],
)


def _sc_body(embq_hbm, idxf_hbm, q_hbm, idx_a, idx_b, rows_v, sem):
    wid = lax.axis_index("s") * NC + lax.axis_index("c")
    base = wid * BPW

    pltpu.sync_copy(idxf_hbm.at[pl.ds(base, 128)], idx_a)
    pltpu.sync_copy(idxf_hbm.at[pl.ds(base + 128, 128)], idx_b)

    cp0 = pltpu.async_copy(embq_hbm.at[idx_a], rows_v.at[pl.ds(0, 128)], sem)
    cp1 = pltpu.async_copy(embq_hbm.at[idx_b], rows_v.at[pl.ds(128, 128)], sem)
    cp0.wait()
    cp1.wait()
    pltpu.sync_copy(rows_v, q_hbm.at[pl.ds(base, BPW)])


_sc_gather = functools.partial(
    pl.kernel,
    mesh=plsc.VectorSubcoreMesh(core_axis_name="c", subcore_axis_name="s"),
    out_type=jax.ShapeDtypeStruct((N, D), jnp.float32),
    scratch_types=[
        pltpu.VMEM((128,), jnp.int32),
        pltpu.VMEM((128,), jnp.int32),
        pltpu.VMEM((BPW, D), jnp.float32),
        pltpu.SemaphoreType.DMA,
    ],
)(_sc_body)


def _fin_body(flat_ref, q_ref, cnt_ref, qst_ref, loss_ref, perp_ref):
    r = pl.program_id(0)
    xb = flat_ref[...]
    qb = q_ref[...].astype(jnp.bfloat16).astype(jnp.float32)
    d = qb - xb
    qst_ref[...] = xb + d

    @pl.when(r == 0)
    def _():
        loss_ref[...] = jnp.zeros((1, 1), jnp.float32)
        perp_ref[...] = jnp.zeros((1, 1), jnp.float32)

    loss_ref[...] += jnp.sum(d * d).reshape(1, 1)

    @pl.when(r == NBLK - 1)
    def _():
        loss_ref[...] = loss_ref[...] * jnp.float32(1.25 / float(N * D))
        cnt = jnp.sum(cnt_ref[...], axis=0)
        p = cnt * jnp.float32(1.0 / N)
        ent = jnp.sum(p * jnp.log(p + jnp.float32(1e-10)))
        perp_ref[...] = jnp.exp(-ent).reshape(1, 1)


_finalize = pl.pallas_call(
    _fin_body,
    grid=(NBLK,),
    in_specs=[
        pl.BlockSpec((RB, D), lambda r: (r, 0)),
        pl.BlockSpec((RB, D), lambda r: (r, 0)),
        pl.BlockSpec((1, K), lambda r: (0, 0)),
    ],
    out_specs=[
        pl.BlockSpec((RB, D), lambda r: (r, 0)),
        pl.BlockSpec((1, 1), lambda r: (0, 0)),
        pl.BlockSpec((1, 1), lambda r: (0, 0)),
    ],
    out_shape=[
        jax.ShapeDtypeStruct((N, D), jnp.float32),
        jax.ShapeDtypeStruct((1, 1), jnp.float32),
        jax.ShapeDtypeStruct((1, 1), jnp.float32),
    ],
)


def kernel(inputs, embedding):
    x = jnp.transpose(inputs, (0, 2, 3, 1))
    input_shape = x.shape
    flat = x.reshape(-1, D)
    distances = (jnp.sum(flat ** 2, axis=1, keepdims=True)
                 + jnp.sum(embedding ** 2, axis=1)
                 - 2.0 * jnp.matmul(flat, embedding.T))
    idx = jnp.argmin(distances, axis=1)

    idx_b, flat_b, emb_b = lax.optimization_barrier((idx, flat, embedding))

    enc, cnt = _onehot(idx_b.reshape(N, 1))
    q = _sc_gather(emb_b, idx_b)

    qst_flat, loss, perp = _finalize(flat_b, q, cnt)
    qst = jnp.transpose(qst_flat.reshape(input_shape), (0, 3, 1, 2))
    return (loss.reshape(()), qst, perp.reshape(()), enc)
